# Initial kernel scaffold; baseline (speedup 1.0000x reference)
import jax
import jax.numpy as jnp
from jax.experimental import pallas as pl


def kernel(keys, values, queries, top_k):
    raise NotImplementedError("write your pallas kernel here")


# speedup vs baseline: 3.4763x; 3.4763x over previous
import functools

import jax
import jax.numpy as jnp
from jax import lax
from jax.experimental import pallas as pl
from jax.experimental.pallas import tpu as pltpu
from jax.experimental.pallas import tpu_sc as plsc

Q = 1024
N = 100000
DK = 256
DV = 256
K = 8
B = 512
NB = 196
NP = NB * B
SEG = 128
SPB = B // SEG
NSEG = NP // SEG
CAND = K * SEG

NC = 2
NS = 16
NW = NC * NS
QPW = Q // NW

_BIG = 2**30


def _k1_body(q_ref, k_ref, d_ref, m_ref):
    i = pl.program_id(0)
    qs = q_ref[...]
    ks = k_ref[...]
    dots = lax.dot_general(qs, ks, (((1,), (1,)), ((), ())),
                           preferred_element_type=jnp.float32)
    qsq = jnp.sum(qs * qs, axis=1, keepdims=True)
    ksq = jnp.sum(ks * ks, axis=1)[None, :]
    d = (qsq + ksq) - 2.0 * dots
    gcol = i * B + lax.broadcasted_iota(jnp.int32, (Q, B), 1)
    d = jnp.where(gcol < N, d, jnp.inf)
    d_ref[...] = d
    m4 = jnp.concatenate(
        [jnp.min(d[:, s * SEG:(s + 1) * SEG], axis=1, keepdims=True)
         for s in range(SPB)], axis=1)
    m_ref[...] = m4.T[None]


_k1 = pl.pallas_call(
    _k1_body,
    grid=(NB,),
    in_specs=[
        pl.BlockSpec((Q, DK), lambda i: (0, 0)),
        pl.BlockSpec((B, DK), lambda i: (i, 0)),
    ],
    out_specs=[
        pl.BlockSpec((Q, B), lambda i: (0, i)),
        pl.BlockSpec((1, SPB, Q), lambda i: (i, 0, 0)),
    ],
    out_shape=[
        jax.ShapeDtypeStruct((Q, NP), jnp.float32),
        jax.ShapeDtypeStruct((NB, SPB, Q), jnp.float32),
    ],
)


def _k1b_body(m_ref, cidx_ref):
    m = m_ref[...]
    sub = lax.broadcasted_iota(jnp.int32, (NSEG, Q), 0)
    qrow = lax.broadcasted_iota(jnp.int32, (1, Q), 1)
    for r in range(K):
        mn = jnp.min(m, axis=0, keepdims=True)
        am = jnp.min(jnp.where(m == mn, sub, _BIG), axis=0, keepdims=True)
        cidx_ref[r:r + 1, :] = qrow * NSEG + am
        m = jnp.where(sub == am, jnp.inf, m)


_k1b = pl.pallas_call(
    _k1b_body,
    out_shape=jax.ShapeDtypeStruct((K, Q), jnp.int32),
)


def _k2_body(d2_hbm, cidx_hbm, c_hbm, idx_v, cand_v, sem):
    w = lax.axis_index("s") * NC + lax.axis_index("c")
    q0 = w * QPW
    pltpu.sync_copy(cidx_hbm, idx_v)
    for r in range(K):
        for half in range(2):
            idx16 = idx_v[r, pl.ds(q0 + 16 * half, 16)]
            pltpu.async_copy(d2_hbm.at[idx16], cand_v, sem).wait()
            pltpu.sync_copy(
                cand_v, c_hbm.at[pl.ds(r * Q + q0 + 16 * half, 16)])


@functools.cache
def _get_k2():
    mesh = plsc.VectorSubcoreMesh(
        core_axis_name="c", subcore_axis_name="s",
        num_cores=NC, num_subcores=NS)
    return pl.kernel(
        _k2_body,
        out_type=jax.ShapeDtypeStruct((Q * K, SEG), jnp.float32),
        mesh=mesh,
        scratch_types=[
            pltpu.VMEM((K, Q), jnp.int32),
            pltpu.VMEM((16, SEG), jnp.float32),
            pltpu.SemaphoreType.DMA,
        ],
    )


def _k3_body(c_ref, cidx_ref, w_ref, i_ref):
    cv = jnp.concatenate([c_ref[r] for r in range(K)], axis=1)
    cidx = cidx_ref[...].T
    qio = lax.broadcasted_iota(jnp.int32, (Q, 1), 0)
    seg = cidx - qio * NSEG
    j = lax.broadcasted_iota(jnp.int32, (Q, SEG), 1)
    gidx = jnp.concatenate(
        [seg[:, r:r + 1] * SEG + j for r in range(K)], axis=1)
    lane = lax.broadcasted_iota(jnp.int32, (Q, CAND), 1)
    dts, its = [], []
    for r in range(K):
        mn = jnp.min(cv, axis=1, keepdims=True)
        am = jnp.min(jnp.where(cv == mn, lane, _BIG), axis=1, keepdims=True)
        sel = lane == am
        gi = jnp.min(jnp.where(sel, gidx, _BIG), axis=1, keepdims=True)
        dts.append(mn)
        its.append(gi)
        cv = jnp.where(sel, jnp.inf, cv)
    d_top = jnp.concatenate(dts, axis=1)
    e = jnp.exp(d_top[:, 0:1] - d_top)
    wgt = e / jnp.sum(e, axis=1, keepdims=True)
    w_ref[...] = jnp.concatenate(
        [jnp.concatenate([wgt[:, r:r + 1]] * 16, axis=1) for r in range(K)],
        axis=1)
    i_ref[...] = jnp.concatenate(its, axis=1)


_k3 = pl.pallas_call(
    _k3_body,
    out_shape=[
        jax.ShapeDtypeStruct((Q, K * 16), jnp.float32),
        jax.ShapeDtypeStruct((Q, K), jnp.int32),
    ],
)


def _k4_body(vals_hbm, i_hbm, wb_hbm, out_hbm, idx_v, rows_v, wb_v, out_v, sem):
    w = lax.axis_index("s") * NC + lax.axis_index("c")
    pltpu.sync_copy(i_hbm.at[pl.ds(2 * w, 2)], idx_v)
    for h in range(2):
        q0 = w * QPW + h * 16
        pltpu.async_copy(vals_hbm.at[idx_v.at[h]], rows_v, sem).wait()
        pltpu.sync_copy(wb_hbm.at[pl.ds(q0, 16)], wb_v)

        def body(qi, _):
            for c in range(DV // 16):
                acc = jnp.zeros((16,), jnp.float32)
                for r in range(K):
                    acc = acc + (wb_v[qi, pl.ds(16 * r, 16)]
                                 * rows_v[qi * K + r, pl.ds(16 * c, 16)])
                out_v[qi, pl.ds(16 * c, 16)] = acc
            return 0

        lax.fori_loop(0, 16, body, 0)
        pltpu.sync_copy(out_v, out_hbm.at[pl.ds(q0, 16)])


@functools.cache
def _get_k4():
    mesh = plsc.VectorSubcoreMesh(
        core_axis_name="c", subcore_axis_name="s",
        num_cores=NC, num_subcores=NS)
    return pl.kernel(
        _k4_body,
        out_type=jax.ShapeDtypeStruct((Q, DV), jnp.float32),
        mesh=mesh,
        scratch_types=[
            pltpu.VMEM((2, 128), jnp.int32),
            pltpu.VMEM((128, DV), jnp.float32),
            pltpu.VMEM((16, 16 * K), jnp.float32),
            pltpu.VMEM((16, DV), jnp.float32),
            pltpu.SemaphoreType.DMA,
        ],
    )


def kernel(keys, values, queries, top_k):
    del top_k
    keys_p = jnp.pad(keys, ((0, NP - N), (0, 0)))
    d, m3 = _k1(queries, keys_p)
    cidx_t = _k1b(m3.reshape(NSEG, Q))
    c = _get_k2()(d.reshape(Q * NSEG, SEG), cidx_t)
    wgt, idx = _k3(c.reshape(K, Q, SEG), cidx_t)
    return _get_k4()(values, idx.reshape(Q * K // 128, 128), wgt)
